# baseline (speedup 1.0000x reference)
import functools

import jax
import jax.numpy as jnp
from jax import lax
from jax.experimental import pallas as pl
from jax.experimental.pallas import tpu as pltpu
from jax.experimental.pallas import tpu_sc as plsc

N_NODES = 10000
N_EDGES = 320000
D_IN = 128
HIDDEN = 256

_NC = 2
_NS = 16
_C = 125
_G = 20
_NCH1 = N_EDGES // (_NC * _NS) // _C
_RPT = N_NODES // _NS
_CW = 16
_C2 = 100
_C2W = D_IN // 2
_G2 = 20
_NCH2 = N_EDGES // _NS // _C2

_mesh = plsc.VectorSubcoreMesh(core_axis_name="c", subcore_axis_name="s")


def _gather(table, sidx, j, buf, sem):
    return pltpu.make_async_copy(table.at[sidx.at[j]], buf, sem)


def _edge_loop(table, acc, sidx, didx, bufs, sems, n_chunks,
               per_chunk_extra=None):
    for b in range(2):
        _gather(table, sidx, b, bufs[b], sems[b]).start()

    def pair(j, inner):
        for b in range(2):
            jj = 2 * j + b
            _gather(table, sidx, jj, bufs[b], sems[b]).wait()
            pltpu.sync_copy(bufs[b], acc.at[didx.at[jj]], add=True)
            if per_chunk_extra is not None:
                per_chunk_extra(jj)

            @pl.when(jj + 2 < n_chunks)
            def _():
                _gather(table, sidx, jj + 2, bufs[b], sems[b]).start()

        return inner

    lax.fori_loop(0, n_chunks // 2, pair, 0)


@functools.partial(
    pl.kernel,
    mesh=_mesh,
    out_type=(
        jax.ShapeDtypeStruct((_NC, N_NODES, D_IN), jnp.float32),
        jax.ShapeDtypeStruct((_NC, N_NODES, _CW), jnp.float32),
    ),
    compiler_params=pltpu.CompilerParams(use_tc_tiling_on_sc=False),
    scratch_types=[
        pltpu.VMEM((_G, _C), jnp.int32),
        pltpu.VMEM((_G, _C), jnp.int32),
        pltpu.VMEM((_C, D_IN), jnp.float32),
        pltpu.VMEM((_C, D_IN), jnp.float32),
        pltpu.VMEM((_C, _CW), jnp.float32),
        pltpu.VMEM_SHARED((N_NODES, D_IN), jnp.float32),
        pltpu.VMEM_SHARED((N_NODES, _CW), jnp.float32),
        pltpu.SemaphoreType.DMA,
        pltpu.SemaphoreType.DMA,
    ],
)
def _sc_aggregate1(x, src, dst, zrows, zcnt, ones, out, outc, sidx, didx,
                   buf0, buf1, ones_v, acc, accc, sem0, sem1):
    c = lax.axis_index("c")
    s = lax.axis_index("s")
    r0 = s * _RPT
    bufs = (buf0, buf1)
    sems = (sem0, sem1)
    pltpu.sync_copy(zrows, acc.at[pl.ds(r0, _RPT)])
    pltpu.sync_copy(zcnt, accc.at[pl.ds(r0, _RPT)])
    pltpu.sync_copy(ones, ones_v)
    plsc.subcore_barrier()

    def ones_scatter(jj):
        pltpu.sync_copy(ones_v, accc.at[didx.at[jj]], add=True)

    def group(g, carry):
        pltpu.sync_copy(src.at[c, s, pl.ds(g * _G, _G)], sidx)
        pltpu.sync_copy(dst.at[c, s, pl.ds(g * _G, _G)], didx)
        _edge_loop(x, acc, sidx, didx, bufs, sems, _G,
                   per_chunk_extra=ones_scatter)
        return carry

    lax.fori_loop(0, _NCH1 // _G, group, 0)
    plsc.subcore_barrier()
    pltpu.sync_copy(acc.at[pl.ds(r0, _RPT)], out.at[c, pl.ds(r0, _RPT)])
    pltpu.sync_copy(accc.at[pl.ds(r0, _RPT)], outc.at[c, pl.ds(r0, _RPT)])


@functools.partial(
    pl.kernel,
    mesh=_mesh,
    out_type=jax.ShapeDtypeStruct((_NC, N_NODES, D_IN), jnp.float32),
    compiler_params=pltpu.CompilerParams(
        use_tc_tiling_on_sc=False, needs_layout_passes=False),
    scratch_types=[
        pltpu.VMEM((_G2, _C2), jnp.int32),
        pltpu.VMEM((_G2, _C2), jnp.int32),
        pltpu.VMEM((_C2, D_IN), jnp.bfloat16),
        pltpu.VMEM((_C2, D_IN), jnp.bfloat16),
        pltpu.VMEM((_C2, D_IN), jnp.float32),
        pltpu.VMEM((_C2, D_IN), jnp.float32),
        pltpu.VMEM_SHARED((N_NODES, D_IN), jnp.float32),
        pltpu.SemaphoreType.DMA,
        pltpu.SemaphoreType.DMA,
        pltpu.SemaphoreType.DMA,
        pltpu.SemaphoreType.DMA,
    ],
)
def _sc_aggregate2(h2p, src, dst, zrows, out, sidx, didx, bbuf0, bbuf1,
                   fbuf0, fbuf1, acc, gsem0, gsem1, ssem0, ssem1):
    c = lax.axis_index("c")
    s = lax.axis_index("s")
    r0 = s * _RPT
    bbufs = (bbuf0, bbuf1)
    fbufs = (fbuf0, fbuf1)
    gsems = (gsem0, gsem1)
    ssems = (ssem0, ssem1)
    table = h2p.at[c]
    pltpu.sync_copy(zrows, acc.at[pl.ds(r0, _RPT)])
    plsc.subcore_barrier()

    def convert(bbuf, fbuf):
        def row(r, carry):
            for w in range(_C2W // 16):
                ab = bbuf[r, pl.ds(32 * w, 32)]
                a, b = plsc.unpack(ab, format=plsc.PackFormat.INTERLEAVED)
                fbuf[r, pl.ds(16 * w, 16)] = a
                fbuf[r, pl.ds(_C2W + 16 * w, 16)] = b
            return carry

        lax.fori_loop(0, _C2, row, 0)

    def group(g, carry):
        pltpu.sync_copy(src.at[s, pl.ds(g * _G2, _G2)], sidx)
        pltpu.sync_copy(dst.at[s, pl.ds(g * _G2, _G2)], didx)
        for b in range(2):
            _gather(table, sidx, b, bbufs[b], gsems[b]).start()

        def pair(j, inner):
            for b in range(2):
                jj = 2 * j + b
                _gather(table, sidx, jj, bbufs[b], gsems[b]).wait()

                @pl.when(jj >= 2)
                def _():
                    pltpu.make_async_copy(
                        fbufs[b], acc.at[didx.at[jj]], ssems[b]).wait()

                convert(bbufs[b], fbufs[b])
                pltpu.make_async_copy(
                    fbufs[b], acc.at[didx.at[jj]], ssems[b]).start(add=True)

                @pl.when(jj + 2 < _G2)
                def _():
                    _gather(table, sidx, jj + 2, bbufs[b], gsems[b]).start()

            return inner

        lax.fori_loop(0, _G2 // 2, pair, carry)
        for b in range(2):
            pltpu.make_async_copy(
                fbufs[b], acc.at[didx.at[0]], ssems[b]).wait()
        return carry

    lax.fori_loop(0, _NCH2 // _G2, group, 0)
    plsc.subcore_barrier()
    pltpu.sync_copy(acc.at[pl.ds(r0, _RPT)], out.at[c, pl.ds(r0, _RPT)])


_ROWS_TC = 1000


def _pack_bf16_pairs(lo, hi):
    a = lax.bitcast_convert_type(lo, jnp.int32)
    b = lax.bitcast_convert_type(hi, jnp.int32)
    ar = a + jnp.int32(0x7FFF) + (lax.shift_right_logical(a, 16) & jnp.int32(1))
    br = b + jnp.int32(0x7FFF) + (lax.shift_right_logical(b, 16) & jnp.int32(1))
    return lax.shift_right_logical(ar, 16) | (br & jnp.int32(-65536))


def _tc_layer1_body(p1_ref, c1_ref, x_ref, w1l_ref, b1l_ref, w1r_ref,
                    h2_ref, h2p_ref):
    summed = p1_ref[0] + p1_ref[1]
    cnt = c1_ref[0, :, 0:1] + c1_ref[1, :, 0:1]
    mean = summed * (1.0 / jnp.maximum(cnt, 1.0))
    out = (
        jnp.dot(mean, w1l_ref[...], preferred_element_type=jnp.float32)
        + jnp.dot(x_ref[...], w1r_ref[...], preferred_element_type=jnp.float32)
        + b1l_ref[...]
    )
    nrm = jnp.sqrt(jnp.sum(out * out, axis=-1, keepdims=True))
    out = out / jnp.maximum(nrm, 1e-12)
    out = jnp.maximum(out, 0.0)
    h2_ref[0] = out[:, :D_IN]
    h2_ref[1] = out[:, D_IN:]
    h2p_ref[0] = _pack_bf16_pairs(out[:, 0:_C2W], out[:, _C2W:D_IN])
    h2p_ref[1] = _pack_bf16_pairs(out[:, D_IN:D_IN + _C2W], out[:, D_IN + _C2W:])


def _tc_layer1(p1, c1, x, w1l, b1l, w1r):
    grid = (N_NODES // _ROWS_TC,)
    return pl.pallas_call(
        _tc_layer1_body,
        grid=grid,
        in_specs=[
            pl.BlockSpec((2, _ROWS_TC, D_IN), lambda i: (0, i, 0)),
            pl.BlockSpec((2, _ROWS_TC, _CW), lambda i: (0, i, 0)),
            pl.BlockSpec((_ROWS_TC, D_IN), lambda i: (i, 0)),
            pl.BlockSpec((D_IN, HIDDEN), lambda i: (0, 0)),
            pl.BlockSpec((1, HIDDEN), lambda i: (0, 0)),
            pl.BlockSpec((D_IN, HIDDEN), lambda i: (0, 0)),
        ],
        out_specs=[
            pl.BlockSpec((2, _ROWS_TC, D_IN), lambda i: (0, i, 0)),
            pl.BlockSpec((2, _ROWS_TC, _C2W), lambda i: (0, i, 0)),
        ],
        out_shape=[
            jax.ShapeDtypeStruct((2, N_NODES, D_IN), jnp.float32),
            jax.ShapeDtypeStruct((2, N_NODES, _C2W), jnp.int32),
        ],
    )(p1, c1, x, w1l, b1l, w1r)


def _tc_layer2_body(m_ref, c1_ref, h2_ref, w2l_ref, b2l_ref, w2r_ref, out_ref):
    cnt = c1_ref[0, :, 0:1] + c1_ref[1, :, 0:1]
    rc = 1.0 / jnp.maximum(cnt, 1.0)
    ma = m_ref[0] * rc
    mb = m_ref[1] * rc
    out = (
        jnp.dot(ma, w2l_ref[:D_IN, :], preferred_element_type=jnp.float32)
        + jnp.dot(mb, w2l_ref[D_IN:, :], preferred_element_type=jnp.float32)
        + jnp.dot(h2_ref[0], w2r_ref[:D_IN, :], preferred_element_type=jnp.float32)
        + jnp.dot(h2_ref[1], w2r_ref[D_IN:, :], preferred_element_type=jnp.float32)
        + b2l_ref[...]
    )
    nrm = jnp.sqrt(jnp.sum(out * out, axis=-1, keepdims=True))
    out_ref[...] = out / jnp.maximum(nrm, 1e-12)


def _tc_layer2(m, c1, h2, w2l, b2l, w2r):
    grid = (N_NODES // _ROWS_TC,)
    return pl.pallas_call(
        _tc_layer2_body,
        grid=grid,
        in_specs=[
            pl.BlockSpec((2, _ROWS_TC, D_IN), lambda i: (0, i, 0)),
            pl.BlockSpec((2, _ROWS_TC, _CW), lambda i: (0, i, 0)),
            pl.BlockSpec((2, _ROWS_TC, D_IN), lambda i: (0, i, 0)),
            pl.BlockSpec((HIDDEN, HIDDEN), lambda i: (0, 0)),
            pl.BlockSpec((1, HIDDEN), lambda i: (0, 0)),
            pl.BlockSpec((HIDDEN, HIDDEN), lambda i: (0, 0)),
        ],
        out_specs=pl.BlockSpec((_ROWS_TC, HIDDEN), lambda i: (i, 0)),
        out_shape=jax.ShapeDtypeStruct((N_NODES, HIDDEN), jnp.float32),
    )(m, c1, h2, w2l, b2l, w2r)


def kernel(x, edge_index, W1l, b1l, W1r, W2l, b2l, W2r):
    src = edge_index[0].astype(jnp.int32)
    dst = edge_index[1].astype(jnp.int32)

    zrows = jnp.zeros((_RPT, D_IN), jnp.float32)
    zcnt = jnp.zeros((_RPT, _CW), jnp.float32)
    ones = jnp.ones((_C, _CW), jnp.float32)

    src1 = src.reshape(_NC, _NS, _NCH1, _C)
    dst1 = dst.reshape(_NC, _NS, _NCH1, _C)
    p1, c1 = _sc_aggregate1(x, src1, dst1, zrows, zcnt, ones)

    h2, h2p = _tc_layer1(p1, c1, x, W1l, b1l.reshape(1, HIDDEN), W1r)

    h2p_bf = lax.bitcast_convert_type(h2p, jnp.bfloat16).reshape(
        _NC, N_NODES, D_IN)
    src2 = src.reshape(_NS, _NCH2, _C2)
    dst2 = dst.reshape(_NS, _NCH2, _C2)
    m = _sc_aggregate2(h2p_bf, src2, dst2, zrows)

    return _tc_layer2(m, c1, h2, W2l, b2l.reshape(1, HIDDEN), W2r)


# speedup vs baseline: 1.0815x; 1.0815x over previous
import functools

import jax
import jax.numpy as jnp
from jax import lax
from jax.experimental import pallas as pl
from jax.experimental.pallas import tpu as pltpu
from jax.experimental.pallas import tpu_sc as plsc

N_NODES = 10000
N_EDGES = 320000
D_IN = 128
HIDDEN = 256

_NC = 2
_NS = 16
_C = 125
_G = 20
_NCH1 = N_EDGES // (_NC * _NS) // _C
_RPT = N_NODES // _NS
_CW = 16
_C2 = 100
_C2W = D_IN // 2
_G2 = 20
_NCH2 = N_EDGES // _NS // _C2

_mesh = plsc.VectorSubcoreMesh(core_axis_name="c", subcore_axis_name="s")


def _gather(table, sidx, j, buf, sem):
    return pltpu.make_async_copy(table.at[sidx.at[j]], buf, sem)


def _edge_loop(table, acc, sidx, didx, bufs, sems, n_chunks,
               per_chunk_extra=None):
    for b in range(2):
        _gather(table, sidx, b, bufs[b], sems[b]).start()

    def pair(j, inner):
        for b in range(2):
            jj = 2 * j + b
            _gather(table, sidx, jj, bufs[b], sems[b]).wait()
            pltpu.sync_copy(bufs[b], acc.at[didx.at[jj]], add=True)
            if per_chunk_extra is not None:
                per_chunk_extra(jj)

            @pl.when(jj + 2 < n_chunks)
            def _():
                _gather(table, sidx, jj + 2, bufs[b], sems[b]).start()

        return inner

    lax.fori_loop(0, n_chunks // 2, pair, 0)


@functools.partial(
    pl.kernel,
    mesh=_mesh,
    out_type=(
        jax.ShapeDtypeStruct((_NC, N_NODES, D_IN), jnp.float32),
        jax.ShapeDtypeStruct((_NC, N_NODES, _CW), jnp.float32),
    ),
    compiler_params=pltpu.CompilerParams(use_tc_tiling_on_sc=False),
    scratch_types=[
        pltpu.VMEM((_G, _C), jnp.int32),
        pltpu.VMEM((_G, _C), jnp.int32),
        pltpu.VMEM((_C, D_IN), jnp.float32),
        pltpu.VMEM((_C, D_IN), jnp.float32),
        pltpu.VMEM((_C, _CW), jnp.float32),
        pltpu.VMEM_SHARED((N_NODES, D_IN), jnp.float32),
        pltpu.VMEM_SHARED((N_NODES, _CW), jnp.float32),
        pltpu.SemaphoreType.DMA,
        pltpu.SemaphoreType.DMA,
    ],
)
def _sc_aggregate1(x, src, dst, zrows, zcnt, ones, out, outc, sidx, didx,
                   buf0, buf1, ones_v, acc, accc, sem0, sem1):
    c = lax.axis_index("c")
    s = lax.axis_index("s")
    r0 = s * _RPT
    bufs = (buf0, buf1)
    sems = (sem0, sem1)
    pltpu.sync_copy(zrows, acc.at[pl.ds(r0, _RPT)])
    pltpu.sync_copy(zcnt, accc.at[pl.ds(r0, _RPT)])
    pltpu.sync_copy(ones, ones_v)
    plsc.subcore_barrier()

    def ones_scatter(jj):
        pltpu.sync_copy(ones_v, accc.at[didx.at[jj]], add=True)

    def group(g, carry):
        pltpu.sync_copy(src.at[c, s, pl.ds(g * _G, _G)], sidx)
        pltpu.sync_copy(dst.at[c, s, pl.ds(g * _G, _G)], didx)
        _edge_loop(x, acc, sidx, didx, bufs, sems, _G,
                   per_chunk_extra=ones_scatter)
        return carry

    lax.fori_loop(0, _NCH1 // _G, group, 0)
    plsc.subcore_barrier()
    pltpu.sync_copy(acc.at[pl.ds(r0, _RPT)], out.at[c, pl.ds(r0, _RPT)])
    pltpu.sync_copy(accc.at[pl.ds(r0, _RPT)], outc.at[c, pl.ds(r0, _RPT)])


@functools.partial(
    pl.kernel,
    mesh=_mesh,
    out_type=jax.ShapeDtypeStruct((_NC, N_NODES, D_IN), jnp.float32),
    compiler_params=pltpu.CompilerParams(
        use_tc_tiling_on_sc=False, needs_layout_passes=False),
    scratch_types=[
        pltpu.VMEM((_G2, _C2), jnp.int32),
        pltpu.VMEM((_G2, _C2), jnp.int32),
        pltpu.VMEM((_C2, _C2W), jnp.int32),
        pltpu.VMEM((_C2, _C2W), jnp.int32),
        pltpu.VMEM((_C2, D_IN), jnp.float32),
        pltpu.VMEM((_C2, D_IN), jnp.float32),
        pltpu.VMEM_SHARED((N_NODES, D_IN), jnp.float32),
        pltpu.SemaphoreType.DMA,
        pltpu.SemaphoreType.DMA,
        pltpu.SemaphoreType.DMA,
        pltpu.SemaphoreType.DMA,
    ],
)
def _sc_aggregate2(h2p, src, dst, zrows, out, sidx, didx, bbuf0, bbuf1,
                   fbuf0, fbuf1, acc, gsem0, gsem1, ssem0, ssem1):
    c = lax.axis_index("c")
    s = lax.axis_index("s")
    r0 = s * _RPT
    bbufs = (bbuf0, bbuf1)
    fbufs = (fbuf0, fbuf1)
    gsems = (gsem0, gsem1)
    ssems = (ssem0, ssem1)
    table = h2p.at[c]
    pltpu.sync_copy(zrows, acc.at[pl.ds(r0, _RPT)])
    plsc.subcore_barrier()

    def convert(bbuf, fbuf):
        mask = jnp.int32(-65536)

        def row(r, carry):
            for w in range(_C2W // 16):
                v = bbuf[r, pl.ds(16 * w, 16)]
                fbuf[r, pl.ds(16 * w, 16)] = plsc.bitcast(
                    lax.shift_left(v, 16), jnp.float32)
                fbuf[r, pl.ds(_C2W + 16 * w, 16)] = plsc.bitcast(
                    v & mask, jnp.float32)
            return carry

        lax.fori_loop(0, _C2, row, 0)

    def group(g, carry):
        pltpu.sync_copy(src.at[s, pl.ds(g * _G2, _G2)], sidx)
        pltpu.sync_copy(dst.at[s, pl.ds(g * _G2, _G2)], didx)
        for b in range(2):
            _gather(table, sidx, b, bbufs[b], gsems[b]).start()

        def pair(j, inner):
            for b in range(2):
                jj = 2 * j + b
                _gather(table, sidx, jj, bbufs[b], gsems[b]).wait()

                @pl.when(jj >= 2)
                def _():
                    pltpu.make_async_copy(
                        fbufs[b], acc.at[didx.at[jj]], ssems[b]).wait()

                convert(bbufs[b], fbufs[b])
                pltpu.make_async_copy(
                    fbufs[b], acc.at[didx.at[jj]], ssems[b]).start(add=True)

                @pl.when(jj + 2 < _G2)
                def _():
                    _gather(table, sidx, jj + 2, bbufs[b], gsems[b]).start()

            return inner

        lax.fori_loop(0, _G2 // 2, pair, carry)
        for b in range(2):
            pltpu.make_async_copy(
                fbufs[b], acc.at[didx.at[0]], ssems[b]).wait()
        return carry

    lax.fori_loop(0, _NCH2 // _G2, group, 0)
    plsc.subcore_barrier()
    pltpu.sync_copy(acc.at[pl.ds(r0, _RPT)], out.at[c, pl.ds(r0, _RPT)])


_ROWS_TC = 1000


def _pack_bf16_pairs(lo, hi):
    a = lax.bitcast_convert_type(lo, jnp.int32)
    b = lax.bitcast_convert_type(hi, jnp.int32)
    ar = a + jnp.int32(0x7FFF) + (lax.shift_right_logical(a, 16) & jnp.int32(1))
    br = b + jnp.int32(0x7FFF) + (lax.shift_right_logical(b, 16) & jnp.int32(1))
    return lax.shift_right_logical(ar, 16) | (br & jnp.int32(-65536))


def _tc_layer1_body(p1_ref, c1_ref, x_ref, w1l_ref, b1l_ref, w1r_ref,
                    h2_ref, h2p_ref):
    summed = p1_ref[0] + p1_ref[1]
    cnt = c1_ref[0, :, 0:1] + c1_ref[1, :, 0:1]
    mean = summed * (1.0 / jnp.maximum(cnt, 1.0))
    out = (
        jnp.dot(mean, w1l_ref[...], preferred_element_type=jnp.float32)
        + jnp.dot(x_ref[...], w1r_ref[...], preferred_element_type=jnp.float32)
        + b1l_ref[...]
    )
    nrm = jnp.sqrt(jnp.sum(out * out, axis=-1, keepdims=True))
    out = out / jnp.maximum(nrm, 1e-12)
    out = jnp.maximum(out, 0.0)
    h2_ref[0] = out[:, :D_IN]
    h2_ref[1] = out[:, D_IN:]
    h2p_ref[0] = _pack_bf16_pairs(out[:, 0:_C2W], out[:, _C2W:D_IN])
    h2p_ref[1] = _pack_bf16_pairs(out[:, D_IN:D_IN + _C2W], out[:, D_IN + _C2W:])


def _tc_layer1(p1, c1, x, w1l, b1l, w1r):
    grid = (N_NODES // _ROWS_TC,)
    return pl.pallas_call(
        _tc_layer1_body,
        grid=grid,
        in_specs=[
            pl.BlockSpec((2, _ROWS_TC, D_IN), lambda i: (0, i, 0)),
            pl.BlockSpec((2, _ROWS_TC, _CW), lambda i: (0, i, 0)),
            pl.BlockSpec((_ROWS_TC, D_IN), lambda i: (i, 0)),
            pl.BlockSpec((D_IN, HIDDEN), lambda i: (0, 0)),
            pl.BlockSpec((1, HIDDEN), lambda i: (0, 0)),
            pl.BlockSpec((D_IN, HIDDEN), lambda i: (0, 0)),
        ],
        out_specs=[
            pl.BlockSpec((2, _ROWS_TC, D_IN), lambda i: (0, i, 0)),
            pl.BlockSpec((2, _ROWS_TC, _C2W), lambda i: (0, i, 0)),
        ],
        out_shape=[
            jax.ShapeDtypeStruct((2, N_NODES, D_IN), jnp.float32),
            jax.ShapeDtypeStruct((2, N_NODES, _C2W), jnp.int32),
        ],
    )(p1, c1, x, w1l, b1l, w1r)


def _tc_layer2_body(m_ref, c1_ref, h2_ref, w2l_ref, b2l_ref, w2r_ref, out_ref):
    cnt = c1_ref[0, :, 0:1] + c1_ref[1, :, 0:1]
    rc = 1.0 / jnp.maximum(cnt, 1.0)
    ma = m_ref[0] * rc
    mb = m_ref[1] * rc
    out = (
        jnp.dot(ma, w2l_ref[:D_IN, :], preferred_element_type=jnp.float32)
        + jnp.dot(mb, w2l_ref[D_IN:, :], preferred_element_type=jnp.float32)
        + jnp.dot(h2_ref[0], w2r_ref[:D_IN, :], preferred_element_type=jnp.float32)
        + jnp.dot(h2_ref[1], w2r_ref[D_IN:, :], preferred_element_type=jnp.float32)
        + b2l_ref[...]
    )
    nrm = jnp.sqrt(jnp.sum(out * out, axis=-1, keepdims=True))
    out_ref[...] = out / jnp.maximum(nrm, 1e-12)


def _tc_layer2(m, c1, h2, w2l, b2l, w2r):
    grid = (N_NODES // _ROWS_TC,)
    return pl.pallas_call(
        _tc_layer2_body,
        grid=grid,
        in_specs=[
            pl.BlockSpec((2, _ROWS_TC, D_IN), lambda i: (0, i, 0)),
            pl.BlockSpec((2, _ROWS_TC, _CW), lambda i: (0, i, 0)),
            pl.BlockSpec((2, _ROWS_TC, D_IN), lambda i: (0, i, 0)),
            pl.BlockSpec((HIDDEN, HIDDEN), lambda i: (0, 0)),
            pl.BlockSpec((1, HIDDEN), lambda i: (0, 0)),
            pl.BlockSpec((HIDDEN, HIDDEN), lambda i: (0, 0)),
        ],
        out_specs=pl.BlockSpec((_ROWS_TC, HIDDEN), lambda i: (i, 0)),
        out_shape=jax.ShapeDtypeStruct((N_NODES, HIDDEN), jnp.float32),
    )(m, c1, h2, w2l, b2l, w2r)


def kernel(x, edge_index, W1l, b1l, W1r, W2l, b2l, W2r):
    src = edge_index[0].astype(jnp.int32)
    dst = edge_index[1].astype(jnp.int32)

    zrows = jnp.zeros((_RPT, D_IN), jnp.float32)
    zcnt = jnp.zeros((_RPT, _CW), jnp.float32)
    ones = jnp.ones((_C, _CW), jnp.float32)

    src1 = src.reshape(_NC, _NS, _NCH1, _C)
    dst1 = dst.reshape(_NC, _NS, _NCH1, _C)
    p1, c1 = _sc_aggregate1(x, src1, dst1, zrows, zcnt, ones)

    h2, h2p = _tc_layer1(p1, c1, x, W1l, b1l.reshape(1, HIDDEN), W1r)

    src2 = src.reshape(_NS, _NCH2, _C2)
    dst2 = dst.reshape(_NS, _NCH2, _C2)
    m = _sc_aggregate2(h2p, src2, dst2, zrows)

    return _tc_layer2(m, c1, h2, W2l, b2l.reshape(1, HIDDEN), W2r)


# speedup vs baseline: 1.6166x; 1.4949x over previous
import functools

import jax
import jax.numpy as jnp
from jax import lax
from jax.experimental import pallas as pl
from jax.experimental.pallas import tpu as pltpu
from jax.experimental.pallas import tpu_sc as plsc

N_NODES = 10000
N_EDGES = 320000
D_IN = 128
HIDDEN = 256

_NC = 2
_NS = 16
_C = 125
_G = 20
_NCH1 = N_EDGES // (_NC * _NS) // _C
_RPT = N_NODES // _NS
_CW = 16
_NCH2 = N_EDGES // _NS // _C

_mesh = plsc.VectorSubcoreMesh(core_axis_name="c", subcore_axis_name="s")


def _gather(table, sidx, j, buf, sem):
    return pltpu.make_async_copy(table.at[sidx.at[j]], buf, sem)


def _edge_loop(table, acc, sidx, didx, bufs, sems, n_chunks,
               per_chunk_extra=None):
    for b in range(2):
        _gather(table, sidx, b, bufs[b], sems[b]).start()

    def pair(j, inner):
        for b in range(2):
            jj = 2 * j + b
            _gather(table, sidx, jj, bufs[b], sems[b]).wait()
            pltpu.sync_copy(bufs[b], acc.at[didx.at[jj]], add=True)
            if per_chunk_extra is not None:
                per_chunk_extra(jj)

            @pl.when(jj + 2 < n_chunks)
            def _():
                _gather(table, sidx, jj + 2, bufs[b], sems[b]).start()

        return inner

    lax.fori_loop(0, n_chunks // 2, pair, 0)


@functools.partial(
    pl.kernel,
    mesh=_mesh,
    out_type=(
        jax.ShapeDtypeStruct((_NC, N_NODES, D_IN), jnp.float32),
        jax.ShapeDtypeStruct((_NC, N_NODES, _CW), jnp.float32),
    ),
    compiler_params=pltpu.CompilerParams(use_tc_tiling_on_sc=False),
    scratch_types=[
        pltpu.VMEM((_G, _C), jnp.int32),
        pltpu.VMEM((_G, _C), jnp.int32),
        pltpu.VMEM((_C, D_IN), jnp.float32),
        pltpu.VMEM((_C, D_IN), jnp.float32),
        pltpu.VMEM((_C, _CW), jnp.float32),
        pltpu.VMEM_SHARED((N_NODES, D_IN), jnp.float32),
        pltpu.VMEM_SHARED((N_NODES, _CW), jnp.float32),
        pltpu.SemaphoreType.DMA,
        pltpu.SemaphoreType.DMA,
    ],
)
def _sc_aggregate1(x, src, dst, zrows, zcnt, ones, out, outc, sidx, didx,
                   buf0, buf1, ones_v, acc, accc, sem0, sem1):
    c = lax.axis_index("c")
    s = lax.axis_index("s")
    r0 = s * _RPT
    bufs = (buf0, buf1)
    sems = (sem0, sem1)
    pltpu.sync_copy(zrows, acc.at[pl.ds(r0, _RPT)])
    pltpu.sync_copy(zcnt, accc.at[pl.ds(r0, _RPT)])
    pltpu.sync_copy(ones, ones_v)
    plsc.subcore_barrier()

    def ones_scatter(jj):
        pltpu.sync_copy(ones_v, accc.at[didx.at[jj]], add=True)

    def group(g, carry):
        pltpu.sync_copy(src.at[c, s, pl.ds(g * _G, _G)], sidx)
        pltpu.sync_copy(dst.at[c, s, pl.ds(g * _G, _G)], didx)
        _edge_loop(x, acc, sidx, didx, bufs, sems, _G,
                   per_chunk_extra=ones_scatter)
        return carry

    lax.fori_loop(0, _NCH1 // _G, group, 0)
    plsc.subcore_barrier()
    pltpu.sync_copy(acc.at[pl.ds(r0, _RPT)], out.at[c, pl.ds(r0, _RPT)])
    pltpu.sync_copy(accc.at[pl.ds(r0, _RPT)], outc.at[c, pl.ds(r0, _RPT)])


@functools.partial(
    pl.kernel,
    mesh=_mesh,
    out_type=jax.ShapeDtypeStruct((_NC, N_NODES, D_IN), jnp.float32),
    compiler_params=pltpu.CompilerParams(use_tc_tiling_on_sc=False),
    scratch_types=[
        pltpu.VMEM((_G, _C), jnp.int32),
        pltpu.VMEM((_G, _C), jnp.int32),
        pltpu.VMEM((_C, D_IN), jnp.float32),
        pltpu.VMEM((_C, D_IN), jnp.float32),
        pltpu.VMEM_SHARED((N_NODES, D_IN), jnp.float32),
        pltpu.SemaphoreType.DMA,
        pltpu.SemaphoreType.DMA,
    ],
)
def _sc_aggregate2(h2, src, dst, zrows, out, sidx, didx, buf0, buf1,
                   acc, sem0, sem1):
    c = lax.axis_index("c")
    s = lax.axis_index("s")
    r0 = s * _RPT
    bufs = (buf0, buf1)
    sems = (sem0, sem1)
    table = h2.at[c]
    pltpu.sync_copy(zrows, acc.at[pl.ds(r0, _RPT)])
    plsc.subcore_barrier()

    def group(g, carry):
        pltpu.sync_copy(src.at[s, pl.ds(g * _G, _G)], sidx)
        pltpu.sync_copy(dst.at[s, pl.ds(g * _G, _G)], didx)
        _edge_loop(table, acc, sidx, didx, bufs, sems, _G)
        return carry

    lax.fori_loop(0, _NCH2 // _G, group, 0)
    plsc.subcore_barrier()
    pltpu.sync_copy(acc.at[pl.ds(r0, _RPT)], out.at[c, pl.ds(r0, _RPT)])


_ROWS_TC = 1000


def _tc_layer1_body(p1_ref, c1_ref, x_ref, w1l_ref, b1l_ref, w1r_ref, h2_ref):
    summed = p1_ref[0] + p1_ref[1]
    cnt = c1_ref[0, :, 0:1] + c1_ref[1, :, 0:1]
    mean = summed * (1.0 / jnp.maximum(cnt, 1.0))
    out = (
        jnp.dot(mean, w1l_ref[...], preferred_element_type=jnp.float32)
        + jnp.dot(x_ref[...], w1r_ref[...], preferred_element_type=jnp.float32)
        + b1l_ref[...]
    )
    nrm = jnp.sqrt(jnp.sum(out * out, axis=-1, keepdims=True))
    out = out / jnp.maximum(nrm, 1e-12)
    out = jnp.maximum(out, 0.0)
    h2_ref[0] = out[:, :D_IN]
    h2_ref[1] = out[:, D_IN:]


def _tc_layer1(p1, c1, x, w1l, b1l, w1r):
    grid = (N_NODES // _ROWS_TC,)
    return pl.pallas_call(
        _tc_layer1_body,
        grid=grid,
        in_specs=[
            pl.BlockSpec((2, _ROWS_TC, D_IN), lambda i: (0, i, 0)),
            pl.BlockSpec((2, _ROWS_TC, _CW), lambda i: (0, i, 0)),
            pl.BlockSpec((_ROWS_TC, D_IN), lambda i: (i, 0)),
            pl.BlockSpec((D_IN, HIDDEN), lambda i: (0, 0)),
            pl.BlockSpec((1, HIDDEN), lambda i: (0, 0)),
            pl.BlockSpec((D_IN, HIDDEN), lambda i: (0, 0)),
        ],
        out_specs=pl.BlockSpec((2, _ROWS_TC, D_IN), lambda i: (0, i, 0)),
        out_shape=jax.ShapeDtypeStruct((2, N_NODES, D_IN), jnp.float32),
    )(p1, c1, x, w1l, b1l, w1r)


def _tc_layer2_body(m_ref, c1_ref, h2_ref, w2l_ref, b2l_ref, w2r_ref, out_ref):
    cnt = c1_ref[0, :, 0:1] + c1_ref[1, :, 0:1]
    rc = 1.0 / jnp.maximum(cnt, 1.0)
    ma = m_ref[0] * rc
    mb = m_ref[1] * rc
    out = (
        jnp.dot(ma, w2l_ref[:D_IN, :], preferred_element_type=jnp.float32)
        + jnp.dot(mb, w2l_ref[D_IN:, :], preferred_element_type=jnp.float32)
        + jnp.dot(h2_ref[0], w2r_ref[:D_IN, :], preferred_element_type=jnp.float32)
        + jnp.dot(h2_ref[1], w2r_ref[D_IN:, :], preferred_element_type=jnp.float32)
        + b2l_ref[...]
    )
    nrm = jnp.sqrt(jnp.sum(out * out, axis=-1, keepdims=True))
    out_ref[...] = out / jnp.maximum(nrm, 1e-12)


def _tc_layer2(m, c1, h2, w2l, b2l, w2r):
    grid = (N_NODES // _ROWS_TC,)
    return pl.pallas_call(
        _tc_layer2_body,
        grid=grid,
        in_specs=[
            pl.BlockSpec((2, _ROWS_TC, D_IN), lambda i: (0, i, 0)),
            pl.BlockSpec((2, _ROWS_TC, _CW), lambda i: (0, i, 0)),
            pl.BlockSpec((2, _ROWS_TC, D_IN), lambda i: (0, i, 0)),
            pl.BlockSpec((HIDDEN, HIDDEN), lambda i: (0, 0)),
            pl.BlockSpec((1, HIDDEN), lambda i: (0, 0)),
            pl.BlockSpec((HIDDEN, HIDDEN), lambda i: (0, 0)),
        ],
        out_specs=pl.BlockSpec((_ROWS_TC, HIDDEN), lambda i: (i, 0)),
        out_shape=jax.ShapeDtypeStruct((N_NODES, HIDDEN), jnp.float32),
    )(m, c1, h2, w2l, b2l, w2r)


def kernel(x, edge_index, W1l, b1l, W1r, W2l, b2l, W2r):
    src = edge_index[0].astype(jnp.int32)
    dst = edge_index[1].astype(jnp.int32)

    zrows = jnp.zeros((_RPT, D_IN), jnp.float32)
    zcnt = jnp.zeros((_RPT, _CW), jnp.float32)
    ones = jnp.ones((_C, _CW), jnp.float32)

    src1 = src.reshape(_NC, _NS, _NCH1, _C)
    dst1 = dst.reshape(_NC, _NS, _NCH1, _C)
    p1, c1 = _sc_aggregate1(x, src1, dst1, zrows, zcnt, ones)

    h2 = _tc_layer1(p1, c1, x, W1l, b1l.reshape(1, HIDDEN), W1r)

    src2 = src.reshape(_NS, _NCH2, _C)
    dst2 = dst.reshape(_NS, _NCH2, _C)
    m = _sc_aggregate2(h2, src2, dst2, zrows)

    return _tc_layer2(m, c1, h2, W2l, b2l.reshape(1, HIDDEN), W2r)


# speedup vs baseline: 1.6707x; 1.0335x over previous
import functools

import jax
import jax.numpy as jnp
from jax import lax
from jax.experimental import pallas as pl
from jax.experimental.pallas import tpu as pltpu
from jax.experimental.pallas import tpu_sc as plsc

N_NODES = 10000
N_EDGES = 320000
D_IN = 128
HIDDEN = 256

_NC = 2
_NS = 16
_C = 125
_G = 20
_NCH1 = N_EDGES // (_NC * _NS) // _C
_RPT = N_NODES // _NS
_CW = 16
_G2 = 40
_NCH2 = N_EDGES // _NS // _C

_mesh = plsc.VectorSubcoreMesh(core_axis_name="c", subcore_axis_name="s")


def _gather(table, sidx, j, buf, sem):
    return pltpu.make_async_copy(table.at[sidx.at[j]], buf, sem)


def _edge_loop(table, acc, sidx, didx, bufs, sems, n_chunks,
               per_chunk_extra=None):
    for b in range(2):
        _gather(table, sidx, b, bufs[b], sems[b]).start()

    def pair(j, inner):
        for b in range(2):
            jj = 2 * j + b
            _gather(table, sidx, jj, bufs[b], sems[b]).wait()
            pltpu.sync_copy(bufs[b], acc.at[didx.at[jj]], add=True)
            if per_chunk_extra is not None:
                per_chunk_extra(jj)

            @pl.when(jj + 2 < n_chunks)
            def _():
                _gather(table, sidx, jj + 2, bufs[b], sems[b]).start()

        return inner

    lax.fori_loop(0, n_chunks // 2, pair, 0)


@functools.partial(
    pl.kernel,
    mesh=_mesh,
    out_type=(
        jax.ShapeDtypeStruct((_NC, N_NODES, D_IN), jnp.float32),
        jax.ShapeDtypeStruct((_NC, N_NODES, _CW), jnp.float32),
    ),
    compiler_params=pltpu.CompilerParams(use_tc_tiling_on_sc=False),
    scratch_types=[
        pltpu.VMEM((_G, _C), jnp.int32),
        pltpu.VMEM((_G, _C), jnp.int32),
        pltpu.VMEM((_C, D_IN), jnp.float32),
        pltpu.VMEM((_C, D_IN), jnp.float32),
        pltpu.VMEM((_C, _CW), jnp.float32),
        pltpu.VMEM_SHARED((N_NODES, D_IN), jnp.float32),
        pltpu.VMEM_SHARED((N_NODES, _CW), jnp.float32),
        pltpu.SemaphoreType.DMA,
        pltpu.SemaphoreType.DMA,
    ],
)
def _sc_aggregate1(x, src, dst, zrows, zcnt, ones, out, outc, sidx, didx,
                   buf0, buf1, ones_v, acc, accc, sem0, sem1):
    c = lax.axis_index("c")
    s = lax.axis_index("s")
    r0 = s * _RPT
    bufs = (buf0, buf1)
    sems = (sem0, sem1)
    pltpu.sync_copy(zrows, acc.at[pl.ds(r0, _RPT)])
    pltpu.sync_copy(zcnt, accc.at[pl.ds(r0, _RPT)])
    pltpu.sync_copy(ones, ones_v)
    plsc.subcore_barrier()

    def ones_scatter(jj):
        pltpu.sync_copy(ones_v, accc.at[didx.at[jj]], add=True)

    def group(g, carry):
        pltpu.sync_copy(src.at[c, s, pl.ds(g * _G, _G)], sidx)
        pltpu.sync_copy(dst.at[c, s, pl.ds(g * _G, _G)], didx)
        _edge_loop(x, acc, sidx, didx, bufs, sems, _G,
                   per_chunk_extra=ones_scatter)
        return carry

    lax.fori_loop(0, _NCH1 // _G, group, 0)
    plsc.subcore_barrier()
    pltpu.sync_copy(acc.at[pl.ds(r0, _RPT)], out.at[c, pl.ds(r0, _RPT)])
    pltpu.sync_copy(accc.at[pl.ds(r0, _RPT)], outc.at[c, pl.ds(r0, _RPT)])


@functools.partial(
    pl.kernel,
    mesh=_mesh,
    out_type=jax.ShapeDtypeStruct((_NC, N_NODES, D_IN), jnp.float32),
    compiler_params=pltpu.CompilerParams(use_tc_tiling_on_sc=False),
    scratch_types=[
        pltpu.VMEM((_G2, _C), jnp.int32),
        pltpu.VMEM((_G2, _C), jnp.int32),
        pltpu.VMEM((_C, D_IN), jnp.float32),
        pltpu.VMEM((_C, D_IN), jnp.float32),
        pltpu.VMEM_SHARED((N_NODES, D_IN), jnp.float32),
        pltpu.SemaphoreType.DMA,
        pltpu.SemaphoreType.DMA,
    ],
)
def _sc_aggregate2(h2, src, dst, zrows, out, sidx, didx, buf0, buf1,
                   acc, sem0, sem1):
    c = lax.axis_index("c")
    s = lax.axis_index("s")
    r0 = s * _RPT
    bufs = (buf0, buf1)
    sems = (sem0, sem1)
    table = h2.at[c]
    pltpu.sync_copy(zrows, acc.at[pl.ds(r0, _RPT)])
    plsc.subcore_barrier()

    def group(g, carry):
        pltpu.sync_copy(src.at[s, pl.ds(g * _G2, _G2)], sidx)
        pltpu.sync_copy(dst.at[s, pl.ds(g * _G2, _G2)], didx)
        _edge_loop(table, acc, sidx, didx, bufs, sems, _G2)
        return carry

    lax.fori_loop(0, _NCH2 // _G2, group, 0)
    plsc.subcore_barrier()
    pltpu.sync_copy(acc.at[pl.ds(r0, _RPT)], out.at[c, pl.ds(r0, _RPT)])


_ROWS_TC = 2000


def _tc_layer1_body(p1_ref, c1_ref, x_ref, w1l_ref, b1l_ref, w1r_ref, h2_ref):
    summed = p1_ref[0] + p1_ref[1]
    cnt = c1_ref[0, :, 0:1] + c1_ref[1, :, 0:1]
    mean = summed * (1.0 / jnp.maximum(cnt, 1.0))
    out = (
        jnp.dot(mean, w1l_ref[...], preferred_element_type=jnp.float32)
        + jnp.dot(x_ref[...], w1r_ref[...], preferred_element_type=jnp.float32)
        + b1l_ref[...]
    )
    nrm = jnp.sqrt(jnp.sum(out * out, axis=-1, keepdims=True))
    out = out / jnp.maximum(nrm, 1e-12)
    out = jnp.maximum(out, 0.0)
    h2_ref[0] = out[:, :D_IN]
    h2_ref[1] = out[:, D_IN:]


def _tc_layer1(p1, c1, x, w1l, b1l, w1r):
    grid = (N_NODES // _ROWS_TC,)
    return pl.pallas_call(
        _tc_layer1_body,
        grid=grid,
        in_specs=[
            pl.BlockSpec((2, _ROWS_TC, D_IN), lambda i: (0, i, 0)),
            pl.BlockSpec((2, _ROWS_TC, _CW), lambda i: (0, i, 0)),
            pl.BlockSpec((_ROWS_TC, D_IN), lambda i: (i, 0)),
            pl.BlockSpec((D_IN, HIDDEN), lambda i: (0, 0)),
            pl.BlockSpec((1, HIDDEN), lambda i: (0, 0)),
            pl.BlockSpec((D_IN, HIDDEN), lambda i: (0, 0)),
        ],
        out_specs=pl.BlockSpec((2, _ROWS_TC, D_IN), lambda i: (0, i, 0)),
        out_shape=jax.ShapeDtypeStruct((2, N_NODES, D_IN), jnp.float32),
    )(p1, c1, x, w1l, b1l, w1r)


def _tc_layer2_body(m_ref, c1_ref, h2_ref, w2l_ref, b2l_ref, w2r_ref, out_ref):
    cnt = c1_ref[0, :, 0:1] + c1_ref[1, :, 0:1]
    rc = 1.0 / jnp.maximum(cnt, 1.0)
    ma = m_ref[0] * rc
    mb = m_ref[1] * rc
    out = (
        jnp.dot(ma, w2l_ref[:D_IN, :], preferred_element_type=jnp.float32)
        + jnp.dot(mb, w2l_ref[D_IN:, :], preferred_element_type=jnp.float32)
        + jnp.dot(h2_ref[0], w2r_ref[:D_IN, :], preferred_element_type=jnp.float32)
        + jnp.dot(h2_ref[1], w2r_ref[D_IN:, :], preferred_element_type=jnp.float32)
        + b2l_ref[...]
    )
    nrm = jnp.sqrt(jnp.sum(out * out, axis=-1, keepdims=True))
    out_ref[...] = out / jnp.maximum(nrm, 1e-12)


def _tc_layer2(m, c1, h2, w2l, b2l, w2r):
    grid = (N_NODES // _ROWS_TC,)
    return pl.pallas_call(
        _tc_layer2_body,
        grid=grid,
        in_specs=[
            pl.BlockSpec((2, _ROWS_TC, D_IN), lambda i: (0, i, 0)),
            pl.BlockSpec((2, _ROWS_TC, _CW), lambda i: (0, i, 0)),
            pl.BlockSpec((2, _ROWS_TC, D_IN), lambda i: (0, i, 0)),
            pl.BlockSpec((HIDDEN, HIDDEN), lambda i: (0, 0)),
            pl.BlockSpec((1, HIDDEN), lambda i: (0, 0)),
            pl.BlockSpec((HIDDEN, HIDDEN), lambda i: (0, 0)),
        ],
        out_specs=pl.BlockSpec((_ROWS_TC, HIDDEN), lambda i: (i, 0)),
        out_shape=jax.ShapeDtypeStruct((N_NODES, HIDDEN), jnp.float32),
    )(m, c1, h2, w2l, b2l, w2r)


def kernel(x, edge_index, W1l, b1l, W1r, W2l, b2l, W2r):
    src = edge_index[0].astype(jnp.int32)
    dst = edge_index[1].astype(jnp.int32)

    zrows = jnp.zeros((_RPT, D_IN), jnp.float32)
    zcnt = jnp.zeros((_RPT, _CW), jnp.float32)
    ones = jnp.ones((_C, _CW), jnp.float32)

    src1 = src.reshape(_NC, _NS, _NCH1, _C)
    dst1 = dst.reshape(_NC, _NS, _NCH1, _C)
    p1, c1 = _sc_aggregate1(x, src1, dst1, zrows, zcnt, ones)

    h2 = _tc_layer1(p1, c1, x, W1l, b1l.reshape(1, HIDDEN), W1r)

    src2 = src.reshape(_NS, _NCH2, _C)
    dst2 = dst.reshape(_NS, _NCH2, _C)
    m = _sc_aggregate2(h2, src2, dst2, zrows)

    return _tc_layer2(m, c1, h2, W2l, b2l.reshape(1, HIDDEN), W2r)
